# Initial kernel scaffold; baseline (speedup 1.0000x reference)
import jax
import jax.numpy as jnp
from jax.experimental import pallas as pl


def kernel(x, edge_index, norm, W0, b0, Wc, bc, Wout, bout):
    raise NotImplementedError("write your pallas kernel here")


# speedup vs baseline: 4.0536x; 4.0536x over previous
import functools

import jax
import jax.numpy as jnp
from jax import lax
from jax.experimental import pallas as pl
from jax.experimental.pallas import tpu as pltpu
from jax.experimental.pallas import tpu_sc as plsc

N = 10000
E = 320000
D = 128

NC = 2
NS = 16
NW = NC * NS

EPW = E // NW
K = 80
CH = EPW // K
NP = 10240
RPT = NP // NS
ZR = 128


def _sc_aggregate(h, src, dst, norm):
    mesh = plsc.VectorSubcoreMesh(core_axis_name="c", subcore_axis_name="s",
                                  num_cores=NC)

    @functools.partial(
        pl.kernel, mesh=mesh,
        out_type=jax.ShapeDtypeStruct((NC, NP, D), jnp.float32),
        scratch_types=[
            pltpu.VMEM((K,), jnp.int32),
            pltpu.VMEM((K,), jnp.int32),
            pltpu.VMEM((K,), jnp.float32),
            pltpu.VMEM((K, D), jnp.float32),
            pltpu.VMEM((ZR, D), jnp.float32),
            pltpu.VMEM_SHARED((NP, D), jnp.float32),
            pltpu.SemaphoreType.DMA,
        ],
    )
    def agg_kernel(h_hbm, src_hbm, dst_hbm, norm_hbm, out_hbm,
                   src_v, dst_v, norm_v, rows_v, zbuf, agg_sh, sem):
        c = lax.axis_index("c")
        s = lax.axis_index("s")
        wid = s * NC + c

        zero16 = jnp.zeros((16,), jnp.float32)

        def zrow(r, _):
            for j in range(D // 16):
                zbuf[r, pl.ds(j * 16, 16)] = zero16
            return 0

        lax.fori_loop(0, ZR, zrow, 0)

        def zcopy(t, _):
            pltpu.sync_copy(zbuf, agg_sh.at[pl.ds(s * RPT + t * ZR, ZR)])
            return 0

        lax.fori_loop(0, RPT // ZR, zcopy, 0)
        plsc.subcore_barrier()

        def chunk(i, _):
            base = wid * EPW + i * K
            pltpu.sync_copy(src_hbm.at[pl.ds(base, K)], src_v)
            pltpu.sync_copy(dst_hbm.at[pl.ds(base, K)], dst_v)
            pltpu.sync_copy(norm_hbm.at[pl.ds(base, K)], norm_v)
            pltpu.async_copy(h_hbm.at[src_v], rows_v, sem).wait()

            def edge16(g, _):
                nv = norm_v[pl.ds(g * 16, 16)]
                for l in range(16):
                    k = g * 16 + l
                    nk = nv[l]
                    for j in range(D // 16):
                        sl = pl.ds(j * 16, 16)
                        rows_v[k, sl] = rows_v[k, sl] * nk
                return 0

            lax.fori_loop(0, K // 16, edge16, 0)
            pltpu.sync_copy(rows_v, agg_sh.at[dst_v], add=True)
            return 0

        lax.fori_loop(0, CH, chunk, 0)
        plsc.subcore_barrier()

        pltpu.sync_copy(agg_sh.at[pl.ds(s * RPT, RPT)],
                        out_hbm.at[c, pl.ds(s * RPT, RPT)])

    return agg_kernel(h, src, dst, norm)


_R = 1000


def _tc_in_kernel(x_ref, w_ref, b_ref, o_ref):
    z = jnp.dot(x_ref[...], w_ref[...], preferred_element_type=jnp.float32)
    o_ref[...] = jnp.maximum(z + b_ref[...], 0.0)


def _tc_in(x, W0, b0):
    return pl.pallas_call(
        _tc_in_kernel,
        grid=(N // _R,),
        in_specs=[
            pl.BlockSpec((_R, D), lambda i: (i, 0)),
            pl.BlockSpec((D, D), lambda i: (0, 0)),
            pl.BlockSpec((1, D), lambda i: (0, 0)),
        ],
        out_specs=pl.BlockSpec((_R, D), lambda i: (i, 0)),
        out_shape=jax.ShapeDtypeStruct((N, D), jnp.float32),
    )(x, W0, b0.reshape(1, D))


def _tc_layer_kernel(p_ref, w_ref, b_ref, x0_ref, o_ref):
    a = p_ref[0] + p_ref[1]
    z = jnp.dot(a, w_ref[...], preferred_element_type=jnp.float32)
    o_ref[...] = jnp.maximum(z + b_ref[...] + x0_ref[...], 0.0)


def _tc_layer(parts, W, b, x0):
    return pl.pallas_call(
        _tc_layer_kernel,
        grid=(N // _R,),
        in_specs=[
            pl.BlockSpec((NC, _R, D), lambda i: (0, i, 0)),
            pl.BlockSpec((D, D), lambda i: (0, 0)),
            pl.BlockSpec((1, D), lambda i: (0, 0)),
            pl.BlockSpec((_R, D), lambda i: (i, 0)),
        ],
        out_specs=pl.BlockSpec((_R, D), lambda i: (i, 0)),
        out_shape=jax.ShapeDtypeStruct((N, D), jnp.float32),
    )(parts, W, b.reshape(1, D), x0)


def _tc_out_kernel(h_ref, w_ref, b_ref, o_ref):
    z = jnp.dot(h_ref[...], w_ref[...], preferred_element_type=jnp.float32)
    z = z + b_ref[...]
    m = jnp.max(z, axis=1, keepdims=True)
    lse = jnp.log(jnp.sum(jnp.exp(z - m), axis=1, keepdims=True))
    o_ref[...] = z - m - lse


def _tc_out(h, Wout, bout):
    dout = Wout.shape[1]
    return pl.pallas_call(
        _tc_out_kernel,
        grid=(N // _R,),
        in_specs=[
            pl.BlockSpec((_R, D), lambda i: (i, 0)),
            pl.BlockSpec((D, dout), lambda i: (0, 0)),
            pl.BlockSpec((1, dout), lambda i: (0, 0)),
        ],
        out_specs=pl.BlockSpec((_R, dout), lambda i: (i, 0)),
        out_shape=jax.ShapeDtypeStruct((N, dout), jnp.float32),
    )(h, Wout, bout.reshape(1, dout))


def kernel(x, edge_index, norm, W0, b0, Wc, bc, Wout, bout):
    src = edge_index[0]
    dst = edge_index[1]
    h = _tc_in(x, W0, b0)
    x0 = h
    for i in range(Wc.shape[0]):
        parts = _sc_aggregate(h, src, dst, norm)
        h = _tc_layer(parts, Wc[i], bc[i], x0)
    return _tc_out(h, Wout, bout)


# speedup vs baseline: 8.8293x; 2.1781x over previous
import functools

import jax
import jax.numpy as jnp
from jax import lax
from jax.experimental import pallas as pl
from jax.experimental.pallas import tpu as pltpu
from jax.experimental.pallas import tpu_sc as plsc

N = 10000
E = 320000
D = 128

NC = 2
NS = 16
NW = NC * NS

EPW = E // NW
K = 80
CH = EPW // K
NP = 10240
RPT = NP // NS


def _sc_aggregate(h, src, dst, norm):
    mesh = plsc.VectorSubcoreMesh(core_axis_name="c", subcore_axis_name="s",
                                  num_cores=NC)

    @functools.partial(
        pl.kernel, mesh=mesh,
        out_type=jax.ShapeDtypeStruct((NC, NP, D), jnp.float32),
        scratch_types=[
            pltpu.VMEM((EPW,), jnp.float32),
            pltpu.VMEM((K,), jnp.int32),
            pltpu.VMEM((K,), jnp.int32),
            pltpu.VMEM((K,), jnp.int32),
            pltpu.VMEM((K,), jnp.int32),
            pltpu.VMEM((K, D), jnp.float32),
            pltpu.VMEM((K, D), jnp.float32),
            pltpu.VMEM_SHARED((NP, D), jnp.float32),
            pltpu.SemaphoreType.DMA,
            pltpu.SemaphoreType.DMA,
            pltpu.SemaphoreType.DMA,
            pltpu.SemaphoreType.DMA,
            pltpu.SemaphoreType.DMA,
            pltpu.SemaphoreType.DMA,
            pltpu.SemaphoreType.DMA,
        ],
    )
    def agg_kernel(h_hbm, src_hbm, dst_hbm, norm_hbm, out_hbm,
                   norm_all, src0, src1, dst0, dst1, rows0, rows1, agg_sh,
                   psem, ssem0, ssem1, dsem0, dsem1, gsem0, gsem1):
        c = lax.axis_index("c")
        s = lax.axis_index("s")
        wid = s * NC + c
        e0 = wid * EPW

        pltpu.async_copy(norm_hbm.at[pl.ds(e0, EPW)], norm_all, psem)
        pltpu.async_copy(src_hbm.at[pl.ds(e0, K)], src0, ssem0)
        pltpu.async_copy(src_hbm.at[pl.ds(e0 + K, K)], src1, ssem1)
        pltpu.async_copy(dst_hbm.at[pl.ds(e0, K)], dst0, dsem0)
        pltpu.async_copy(dst_hbm.at[pl.ds(e0 + K, K)], dst1, dsem1)

        zero16 = jnp.zeros((16,), jnp.float32)

        def zrow(r, _):
            for j in range(D // 16):
                rows0[r, pl.ds(j * 16, 16)] = zero16
            return 0

        lax.fori_loop(0, K, zrow, 0)

        def zcopy(t, _):
            pltpu.sync_copy(rows0, agg_sh.at[pl.ds(s * RPT + t * K, K)])
            return 0

        lax.fori_loop(0, RPT // K, zcopy, 0)
        plsc.subcore_barrier()

        pltpu.make_async_copy(norm_hbm.at[pl.ds(e0, EPW)], norm_all, psem).wait()
        pltpu.make_async_copy(src_hbm.at[pl.ds(e0, K)], src0, ssem0).wait()
        pltpu.async_copy(h_hbm.at[src0], rows0, gsem0)

        bufs = ((src0, ssem0, dst0, dsem0, rows0, gsem0),
                (src1, ssem1, dst1, dsem1, rows1, gsem1))

        def pair(t, _):
            for par in range(2):
                i = 2 * t + par
                src_c, ssem_c, dst_c, dsem_c, rows_c, gsem_c = bufs[par]
                src_n, ssem_n, dst_n, dsem_n, rows_n, gsem_n = bufs[1 - par]

                @pl.when(i + 1 < CH)
                def _():
                    pltpu.make_async_copy(
                        src_hbm.at[pl.ds(e0 + (i + 1) * K, K)],
                        src_n, ssem_n).wait()
                    pltpu.async_copy(h_hbm.at[src_n], rows_n, gsem_n)

                @pl.when(i < CH)
                def _():
                    pltpu.make_async_copy(
                        h_hbm.at[src_c], rows_c, gsem_c).wait()

                    def edge16(g, _):
                        nv = norm_all[pl.ds(i * K + g * 16, 16)]
                        for l in range(16):
                            k = g * 16 + l
                            nk = nv[l]
                            for j in range(D // 16):
                                sl = pl.ds(j * 16, 16)
                                rows_c[k, sl] = rows_c[k, sl] * nk
                        return 0

                    lax.fori_loop(0, K // 16, edge16, 0)
                    pltpu.make_async_copy(
                        dst_hbm.at[pl.ds(e0 + i * K, K)],
                        dst_c, dsem_c).wait()
                    pltpu.sync_copy(rows_c, agg_sh.at[dst_c], add=True)

                    @pl.when(i + 2 < CH)
                    def _():
                        pltpu.async_copy(
                            src_hbm.at[pl.ds(e0 + (i + 2) * K, K)],
                            src_c, ssem_c)
                        pltpu.async_copy(
                            dst_hbm.at[pl.ds(e0 + (i + 2) * K, K)],
                            dst_c, dsem_c)

            return 0

        lax.fori_loop(0, (CH + 1) // 2, pair, 0)
        plsc.subcore_barrier()

        pltpu.sync_copy(agg_sh.at[pl.ds(s * RPT, RPT)],
                        out_hbm.at[c, pl.ds(s * RPT, RPT)])

    return agg_kernel(h, src, dst, norm)


_R = 1000


def _tc_in_kernel(x_ref, w_ref, b_ref, o_ref):
    z = jnp.dot(x_ref[...], w_ref[...], preferred_element_type=jnp.float32)
    o_ref[...] = jnp.maximum(z + b_ref[...], 0.0)


def _tc_in(x, W0, b0):
    return pl.pallas_call(
        _tc_in_kernel,
        grid=(N // _R,),
        in_specs=[
            pl.BlockSpec((_R, D), lambda i: (i, 0)),
            pl.BlockSpec((D, D), lambda i: (0, 0)),
            pl.BlockSpec((1, D), lambda i: (0, 0)),
        ],
        out_specs=pl.BlockSpec((_R, D), lambda i: (i, 0)),
        out_shape=jax.ShapeDtypeStruct((N, D), jnp.float32),
    )(x, W0, b0.reshape(1, D))


def _tc_layer_kernel(p_ref, w_ref, b_ref, x0_ref, o_ref):
    a = p_ref[0] + p_ref[1]
    z = jnp.dot(a, w_ref[...], preferred_element_type=jnp.float32)
    o_ref[...] = jnp.maximum(z + b_ref[...] + x0_ref[...], 0.0)


def _tc_layer(parts, W, b, x0):
    return pl.pallas_call(
        _tc_layer_kernel,
        grid=(N // _R,),
        in_specs=[
            pl.BlockSpec((NC, _R, D), lambda i: (0, i, 0)),
            pl.BlockSpec((D, D), lambda i: (0, 0)),
            pl.BlockSpec((1, D), lambda i: (0, 0)),
            pl.BlockSpec((_R, D), lambda i: (i, 0)),
        ],
        out_specs=pl.BlockSpec((_R, D), lambda i: (i, 0)),
        out_shape=jax.ShapeDtypeStruct((N, D), jnp.float32),
    )(parts, W, b.reshape(1, D), x0)


def _tc_out_kernel(h_ref, w_ref, b_ref, o_ref):
    z = jnp.dot(h_ref[...], w_ref[...], preferred_element_type=jnp.float32)
    z = z + b_ref[...]
    m = jnp.max(z, axis=1, keepdims=True)
    lse = jnp.log(jnp.sum(jnp.exp(z - m), axis=1, keepdims=True))
    o_ref[...] = z - m - lse


def _tc_out(h, Wout, bout):
    dout = Wout.shape[1]
    return pl.pallas_call(
        _tc_out_kernel,
        grid=(N // _R,),
        in_specs=[
            pl.BlockSpec((_R, D), lambda i: (i, 0)),
            pl.BlockSpec((D, dout), lambda i: (0, 0)),
            pl.BlockSpec((1, dout), lambda i: (0, 0)),
        ],
        out_specs=pl.BlockSpec((_R, dout), lambda i: (i, 0)),
        out_shape=jax.ShapeDtypeStruct((N, dout), jnp.float32),
    )(h, Wout, bout.reshape(1, dout))


def kernel(x, edge_index, norm, W0, b0, Wc, bc, Wout, bout):
    src = edge_index[0]
    dst = edge_index[1]
    h = _tc_in(x, W0, b0)
    x0 = h
    for i in range(Wc.shape[0]):
        parts = _sc_aggregate(h, src, dst, norm)
        h = _tc_layer(parts, Wc[i], bc[i], x0)
    return _tc_out(h, Wout, bout)


# speedup vs baseline: 10.6334x; 1.2043x over previous
import functools

import jax
import jax.numpy as jnp
from jax import lax
from jax.experimental import pallas as pl
from jax.experimental.pallas import tpu as pltpu
from jax.experimental.pallas import tpu_sc as plsc

N = 10000
E = 320000
D = 128

NC = 2
NS = 16
NW = NC * NS

EPW = E // NW
K = 80
CH = EPW // K
NP = 10240
RPT = NP // NS


def _sc_aggregate(h, src, dst, norm):
    mesh = plsc.VectorSubcoreMesh(core_axis_name="c", subcore_axis_name="s",
                                  num_cores=NC)

    @functools.partial(
        pl.kernel, mesh=mesh,
        out_type=jax.ShapeDtypeStruct((NC, NP, D), jnp.float32),
        scratch_types=[
            pltpu.VMEM((EPW,), jnp.float32),
            pltpu.VMEM((K,), jnp.int32),
            pltpu.VMEM((K,), jnp.int32),
            pltpu.VMEM((K,), jnp.int32),
            pltpu.VMEM((K,), jnp.int32),
            pltpu.VMEM((K,), jnp.int32),
            pltpu.VMEM((K,), jnp.int32),
            pltpu.VMEM((K, D), jnp.float32),
            pltpu.VMEM((K, D), jnp.float32),
            pltpu.VMEM((K, D), jnp.float32),
            pltpu.VMEM_SHARED((NP, D), jnp.float32),
            pltpu.SemaphoreType.DMA,
            pltpu.SemaphoreType.DMA,
            pltpu.SemaphoreType.DMA,
            pltpu.SemaphoreType.DMA,
            pltpu.SemaphoreType.DMA,
            pltpu.SemaphoreType.DMA,
            pltpu.SemaphoreType.DMA,
            pltpu.SemaphoreType.DMA,
            pltpu.SemaphoreType.DMA,
            pltpu.SemaphoreType.DMA,
            pltpu.SemaphoreType.DMA,
            pltpu.SemaphoreType.DMA,
            pltpu.SemaphoreType.DMA,
        ],
    )
    def agg_kernel(h_hbm, src_hbm, dst_hbm, norm_hbm, out_hbm,
                   norm_all, src0, src1, src2, dst0, dst1, dst2,
                   rows0, rows1, rows2, agg_sh,
                   psem, ssem0, ssem1, ssem2, dsem0, dsem1, dsem2,
                   gsem0, gsem1, gsem2, csem0, csem1, csem2):
        c = lax.axis_index("c")
        s = lax.axis_index("s")
        wid = s * NC + c
        e0 = wid * EPW

        pltpu.async_copy(norm_hbm.at[pl.ds(e0, EPW)], norm_all, psem)
        pltpu.async_copy(src_hbm.at[pl.ds(e0, K)], src0, ssem0)
        pltpu.async_copy(src_hbm.at[pl.ds(e0 + K, K)], src1, ssem1)
        pltpu.async_copy(src_hbm.at[pl.ds(e0 + 2 * K, K)], src2, ssem2)
        pltpu.async_copy(dst_hbm.at[pl.ds(e0, K)], dst0, dsem0)
        pltpu.async_copy(dst_hbm.at[pl.ds(e0 + K, K)], dst1, dsem1)
        pltpu.async_copy(dst_hbm.at[pl.ds(e0 + 2 * K, K)], dst2, dsem2)

        zero16 = jnp.zeros((16,), jnp.float32)

        def zrow(r, _):
            for j in range(D // 16):
                rows0[r, pl.ds(j * 16, 16)] = zero16
            return 0

        lax.fori_loop(0, K, zrow, 0)

        def zcopy(t, _):
            pltpu.sync_copy(rows0, agg_sh.at[pl.ds(s * RPT + t * K, K)])
            return 0

        lax.fori_loop(0, RPT // K, zcopy, 0)
        plsc.subcore_barrier()

        pltpu.make_async_copy(norm_hbm.at[pl.ds(e0, EPW)], norm_all, psem).wait()
        pltpu.make_async_copy(src_hbm.at[pl.ds(e0, K)], src0, ssem0).wait()
        pltpu.async_copy(h_hbm.at[src0], rows0, gsem0)

        bufs = ((src0, ssem0, dst0, dsem0, rows0, gsem0, csem0),
                (src1, ssem1, dst1, dsem1, rows1, gsem1, csem1),
                (src2, ssem2, dst2, dsem2, rows2, gsem2, csem2))

        def triple(t, _):
            for r in range(3):
                i = 3 * t + r
                src_c, ssem_c, dst_c, dsem_c, rows_c, gsem_c, csem_c = bufs[r]
                (src_n, ssem_n, dst_n, dsem_n,
                 rows_n, gsem_n, csem_n) = bufs[(r + 1) % 3]
                (src_n2, ssem_n2, dst_n2, dsem_n2,
                 rows_n2, gsem_n2, csem_n2) = bufs[(r + 2) % 3]

                @pl.when(i + 1 < CH)
                def _():
                    pltpu.make_async_copy(
                        src_hbm.at[pl.ds(e0 + (i + 1) * K, K)],
                        src_n, ssem_n).wait()
                    pltpu.async_copy(h_hbm.at[src_n], rows_n, gsem_n)

                @pl.when(i < CH)
                def _():
                    pltpu.make_async_copy(
                        h_hbm.at[src_c], rows_c, gsem_c).wait()

                    def edge16(g, _):
                        nv = norm_all[pl.ds(i * K + g * 16, 16)]
                        for l in range(16):
                            k = g * 16 + l
                            nk = nv[l]
                            for j in range(D // 16):
                                sl = pl.ds(j * 16, 16)
                                rows_c[k, sl] = rows_c[k, sl] * nk
                        return 0

                    lax.fori_loop(0, K // 16, edge16, 0)
                    pltpu.make_async_copy(
                        dst_hbm.at[pl.ds(e0 + i * K, K)],
                        dst_c, dsem_c).wait()
                    pltpu.async_copy(rows_c, agg_sh.at[dst_c], csem_c,
                                     add=True)

                @pl.when(jnp.logical_and(i >= 1, i + 2 < CH))
                def _():
                    pltpu.make_async_copy(
                        rows_n2, agg_sh.at[dst_n2], csem_n2).wait()
                    pltpu.async_copy(
                        src_hbm.at[pl.ds(e0 + (i + 2) * K, K)],
                        src_n2, ssem_n2)
                    pltpu.async_copy(
                        dst_hbm.at[pl.ds(e0 + (i + 2) * K, K)],
                        dst_n2, dsem_n2)

            return 0

        lax.fori_loop(0, (CH + 2) // 3, triple, 0)

        pltpu.make_async_copy(rows0, agg_sh.at[dst0], csem0).wait()
        pltpu.make_async_copy(rows1, agg_sh.at[dst1], csem1).wait()
        pltpu.make_async_copy(rows2, agg_sh.at[dst2], csem2).wait()
        plsc.subcore_barrier()

        pltpu.sync_copy(agg_sh.at[pl.ds(s * RPT, RPT)],
                        out_hbm.at[c, pl.ds(s * RPT, RPT)])

    return agg_kernel(h, src, dst, norm)


_R = 1000


def _tc_in_kernel(x_ref, w_ref, b_ref, o_ref):
    z = jnp.dot(x_ref[...], w_ref[...], preferred_element_type=jnp.float32)
    o_ref[...] = jnp.maximum(z + b_ref[...], 0.0)


def _tc_in(x, W0, b0):
    return pl.pallas_call(
        _tc_in_kernel,
        grid=(N // _R,),
        in_specs=[
            pl.BlockSpec((_R, D), lambda i: (i, 0)),
            pl.BlockSpec((D, D), lambda i: (0, 0)),
            pl.BlockSpec((1, D), lambda i: (0, 0)),
        ],
        out_specs=pl.BlockSpec((_R, D), lambda i: (i, 0)),
        out_shape=jax.ShapeDtypeStruct((N, D), jnp.float32),
    )(x, W0, b0.reshape(1, D))


def _tc_layer_kernel(p_ref, w_ref, b_ref, x0_ref, o_ref):
    a = p_ref[0] + p_ref[1]
    z = jnp.dot(a, w_ref[...], preferred_element_type=jnp.float32)
    o_ref[...] = jnp.maximum(z + b_ref[...] + x0_ref[...], 0.0)


def _tc_layer(parts, W, b, x0):
    return pl.pallas_call(
        _tc_layer_kernel,
        grid=(N // _R,),
        in_specs=[
            pl.BlockSpec((NC, _R, D), lambda i: (0, i, 0)),
            pl.BlockSpec((D, D), lambda i: (0, 0)),
            pl.BlockSpec((1, D), lambda i: (0, 0)),
            pl.BlockSpec((_R, D), lambda i: (i, 0)),
        ],
        out_specs=pl.BlockSpec((_R, D), lambda i: (i, 0)),
        out_shape=jax.ShapeDtypeStruct((N, D), jnp.float32),
    )(parts, W, b.reshape(1, D), x0)


def _tc_out_kernel(h_ref, w_ref, b_ref, o_ref):
    z = jnp.dot(h_ref[...], w_ref[...], preferred_element_type=jnp.float32)
    z = z + b_ref[...]
    m = jnp.max(z, axis=1, keepdims=True)
    lse = jnp.log(jnp.sum(jnp.exp(z - m), axis=1, keepdims=True))
    o_ref[...] = z - m - lse


def _tc_out(h, Wout, bout):
    dout = Wout.shape[1]
    return pl.pallas_call(
        _tc_out_kernel,
        grid=(N // _R,),
        in_specs=[
            pl.BlockSpec((_R, D), lambda i: (i, 0)),
            pl.BlockSpec((D, dout), lambda i: (0, 0)),
            pl.BlockSpec((1, dout), lambda i: (0, 0)),
        ],
        out_specs=pl.BlockSpec((_R, dout), lambda i: (i, 0)),
        out_shape=jax.ShapeDtypeStruct((N, dout), jnp.float32),
    )(h, Wout, bout.reshape(1, dout))


def kernel(x, edge_index, norm, W0, b0, Wc, bc, Wout, bout):
    src = edge_index[0]
    dst = edge_index[1]
    h = _tc_in(x, W0, b0)
    x0 = h
    for i in range(Wc.shape[0]):
        parts = _sc_aggregate(h, src, dst, norm)
        h = _tc_layer(parts, Wc[i], bc[i], x0)
    return _tc_out(h, Wout, bout)


# speedup vs baseline: 11.3049x; 1.0632x over previous
import functools

import jax
import jax.numpy as jnp
from jax import lax
from jax.experimental import pallas as pl
from jax.experimental.pallas import tpu as pltpu
from jax.experimental.pallas import tpu_sc as plsc

N = 10000
E = 320000
D = 128

NC = 2
NS = 16
NW = NC * NS

EPW0 = E // NW
K = 112
PAD = 80
EPW = EPW0 + PAD
CH = EPW // K
NP = 10240
RPT = NP // NS


def _sc_aggregate(h, src, dst, norm):
    mesh = plsc.VectorSubcoreMesh(core_axis_name="c", subcore_axis_name="s",
                                  num_cores=NC)

    @functools.partial(
        pl.kernel, mesh=mesh,
        out_type=jax.ShapeDtypeStruct((NC, NP, D), jnp.float32),
        scratch_types=[
            pltpu.VMEM((K,), jnp.int32),
            pltpu.VMEM((K,), jnp.int32),
            pltpu.VMEM((K,), jnp.int32),
            pltpu.VMEM((K,), jnp.int32),
            pltpu.VMEM((K,), jnp.int32),
            pltpu.VMEM((K,), jnp.int32),
            pltpu.VMEM((K,), jnp.float32),
            pltpu.VMEM((K,), jnp.float32),
            pltpu.VMEM((K,), jnp.float32),
            pltpu.VMEM((K, D), jnp.float32),
            pltpu.VMEM((K, D), jnp.float32),
            pltpu.VMEM((K, D), jnp.float32),
            pltpu.VMEM_SHARED((NP, D), jnp.float32),
            pltpu.SemaphoreType.DMA,
            pltpu.SemaphoreType.DMA,
            pltpu.SemaphoreType.DMA,
            pltpu.SemaphoreType.DMA,
            pltpu.SemaphoreType.DMA,
            pltpu.SemaphoreType.DMA,
            pltpu.SemaphoreType.DMA,
            pltpu.SemaphoreType.DMA,
            pltpu.SemaphoreType.DMA,
            pltpu.SemaphoreType.DMA,
            pltpu.SemaphoreType.DMA,
            pltpu.SemaphoreType.DMA,
            pltpu.SemaphoreType.DMA,
            pltpu.SemaphoreType.DMA,
            pltpu.SemaphoreType.DMA,
        ],
    )
    def agg_kernel(h_hbm, src_hbm, dst_hbm, norm_hbm, out_hbm,
                   src0, src1, src2, dst0, dst1, dst2,
                   norm0, norm1, norm2, rows0, rows1, rows2, agg_sh,
                   ssem0, ssem1, ssem2, dsem0, dsem1, dsem2,
                   nsem0, nsem1, nsem2,
                   gsem0, gsem1, gsem2, csem0, csem1, csem2):
        c = lax.axis_index("c")
        s = lax.axis_index("s")
        wid = s * NC + c
        e0 = wid * EPW

        pltpu.async_copy(src_hbm.at[pl.ds(e0, K)], src0, ssem0)
        pltpu.async_copy(src_hbm.at[pl.ds(e0 + K, K)], src1, ssem1)
        pltpu.async_copy(src_hbm.at[pl.ds(e0 + 2 * K, K)], src2, ssem2)
        pltpu.async_copy(dst_hbm.at[pl.ds(e0, K)], dst0, dsem0)
        pltpu.async_copy(dst_hbm.at[pl.ds(e0 + K, K)], dst1, dsem1)
        pltpu.async_copy(dst_hbm.at[pl.ds(e0 + 2 * K, K)], dst2, dsem2)
        pltpu.async_copy(norm_hbm.at[pl.ds(e0, K)], norm0, nsem0)
        pltpu.async_copy(norm_hbm.at[pl.ds(e0 + K, K)], norm1, nsem1)
        pltpu.async_copy(norm_hbm.at[pl.ds(e0 + 2 * K, K)], norm2, nsem2)

        zero16 = jnp.zeros((16,), jnp.float32)

        def zrow(r, _):
            for j in range(D // 16):
                rows0[r, pl.ds(j * 16, 16)] = zero16
            return 0

        lax.fori_loop(0, 80, zrow, 0)

        def zcopy(t, _):
            pltpu.sync_copy(rows0.at[pl.ds(0, 80)],
                            agg_sh.at[pl.ds(s * RPT + t * 80, 80)])
            return 0

        lax.fori_loop(0, RPT // 80, zcopy, 0)
        plsc.subcore_barrier()

        pltpu.make_async_copy(src_hbm.at[pl.ds(e0, K)], src0, ssem0).wait()
        pltpu.async_copy(h_hbm.at[src0], rows0, gsem0)

        bufs = ((src0, ssem0, dst0, dsem0, norm0, nsem0, rows0, gsem0, csem0),
                (src1, ssem1, dst1, dsem1, norm1, nsem1, rows1, gsem1, csem1),
                (src2, ssem2, dst2, dsem2, norm2, nsem2, rows2, gsem2, csem2))

        def triple(t, _):
            for r in range(3):
                i = 3 * t + r
                (src_c, ssem_c, dst_c, dsem_c, norm_c, nsem_c,
                 rows_c, gsem_c, csem_c) = bufs[r]
                (src_n, ssem_n, dst_n, dsem_n, norm_n, nsem_n,
                 rows_n, gsem_n, csem_n) = bufs[(r + 1) % 3]
                (src_n2, ssem_n2, dst_n2, dsem_n2, norm_n2, nsem_n2,
                 rows_n2, gsem_n2, csem_n2) = bufs[(r + 2) % 3]

                @pl.when(i + 1 < CH)
                def _():
                    pltpu.make_async_copy(
                        src_hbm.at[pl.ds(e0 + (i + 1) * K, K)],
                        src_n, ssem_n).wait()
                    pltpu.async_copy(h_hbm.at[src_n], rows_n, gsem_n)

                @pl.when(i < CH)
                def _():
                    pltpu.make_async_copy(
                        norm_hbm.at[pl.ds(e0 + i * K, K)],
                        norm_c, nsem_c).wait()
                    pltpu.make_async_copy(
                        h_hbm.at[src_c], rows_c, gsem_c).wait()

                    def edge16(g, _):
                        nv = norm_c[pl.ds(g * 16, 16)]
                        for l in range(16):
                            k = g * 16 + l
                            nk = nv[l]
                            for j in range(D // 16):
                                sl = pl.ds(j * 16, 16)
                                rows_c[k, sl] = rows_c[k, sl] * nk
                        return 0

                    lax.fori_loop(0, K // 16, edge16, 0)
                    pltpu.make_async_copy(
                        dst_hbm.at[pl.ds(e0 + i * K, K)],
                        dst_c, dsem_c).wait()
                    pltpu.async_copy(rows_c, agg_sh.at[dst_c], csem_c,
                                     add=True)

                @pl.when(jnp.logical_and(i >= 1, i + 2 < CH))
                def _():
                    pltpu.make_async_copy(
                        rows_n2, agg_sh.at[dst_n2], csem_n2).wait()
                    pltpu.async_copy(
                        src_hbm.at[pl.ds(e0 + (i + 2) * K, K)],
                        src_n2, ssem_n2)
                    pltpu.async_copy(
                        dst_hbm.at[pl.ds(e0 + (i + 2) * K, K)],
                        dst_n2, dsem_n2)
                    pltpu.async_copy(
                        norm_hbm.at[pl.ds(e0 + (i + 2) * K, K)],
                        norm_n2, nsem_n2)

            return 0

        lax.fori_loop(0, (CH + 2) // 3, triple, 0)

        pltpu.make_async_copy(rows0, agg_sh.at[dst0], csem0).wait()
        pltpu.make_async_copy(rows1, agg_sh.at[dst1], csem1).wait()
        pltpu.make_async_copy(rows2, agg_sh.at[dst2], csem2).wait()
        plsc.subcore_barrier()

        pltpu.sync_copy(agg_sh.at[pl.ds(s * RPT, RPT)],
                        out_hbm.at[c, pl.ds(s * RPT, RPT)])

    return agg_kernel(h, src, dst, norm)


_R = 1000


def _tc_in_kernel(x_ref, w_ref, b_ref, o_ref):
    z = jnp.dot(x_ref[...], w_ref[...], preferred_element_type=jnp.float32)
    o_ref[...] = jnp.maximum(z + b_ref[...], 0.0)


def _tc_in(x, W0, b0):
    return pl.pallas_call(
        _tc_in_kernel,
        grid=(N // _R,),
        in_specs=[
            pl.BlockSpec((_R, D), lambda i: (i, 0)),
            pl.BlockSpec((D, D), lambda i: (0, 0)),
            pl.BlockSpec((1, D), lambda i: (0, 0)),
        ],
        out_specs=pl.BlockSpec((_R, D), lambda i: (i, 0)),
        out_shape=jax.ShapeDtypeStruct((N, D), jnp.float32),
    )(x, W0, b0.reshape(1, D))


def _tc_layer_kernel(p_ref, w_ref, b_ref, x0_ref, o_ref):
    a = p_ref[0] + p_ref[1]
    z = jnp.dot(a, w_ref[...], preferred_element_type=jnp.float32)
    o_ref[...] = jnp.maximum(z + b_ref[...] + x0_ref[...], 0.0)


def _tc_layer(parts, W, b, x0):
    return pl.pallas_call(
        _tc_layer_kernel,
        grid=(N // _R,),
        in_specs=[
            pl.BlockSpec((NC, _R, D), lambda i: (0, i, 0)),
            pl.BlockSpec((D, D), lambda i: (0, 0)),
            pl.BlockSpec((1, D), lambda i: (0, 0)),
            pl.BlockSpec((_R, D), lambda i: (i, 0)),
        ],
        out_specs=pl.BlockSpec((_R, D), lambda i: (i, 0)),
        out_shape=jax.ShapeDtypeStruct((N, D), jnp.float32),
    )(parts, W, b.reshape(1, D), x0)


def _tc_out_kernel(h_ref, w_ref, b_ref, o_ref):
    z = jnp.dot(h_ref[...], w_ref[...], preferred_element_type=jnp.float32)
    z = z + b_ref[...]
    m = jnp.max(z, axis=1, keepdims=True)
    lse = jnp.log(jnp.sum(jnp.exp(z - m), axis=1, keepdims=True))
    o_ref[...] = z - m - lse


def _tc_out(h, Wout, bout):
    dout = Wout.shape[1]
    return pl.pallas_call(
        _tc_out_kernel,
        grid=(N // _R,),
        in_specs=[
            pl.BlockSpec((_R, D), lambda i: (i, 0)),
            pl.BlockSpec((D, dout), lambda i: (0, 0)),
            pl.BlockSpec((1, dout), lambda i: (0, 0)),
        ],
        out_specs=pl.BlockSpec((_R, dout), lambda i: (i, 0)),
        out_shape=jax.ShapeDtypeStruct((N, dout), jnp.float32),
    )(h, Wout, bout.reshape(1, dout))


def kernel(x, edge_index, norm, W0, b0, Wc, bc, Wout, bout):
    spread = (jnp.arange(PAD, dtype=jnp.int32) * 911) % N
    pad_blk = jnp.broadcast_to(spread, (NW, PAD))
    src = jnp.concatenate(
        [edge_index[0].reshape(NW, EPW0), pad_blk], axis=1).reshape(-1)
    dst = jnp.concatenate(
        [edge_index[1].reshape(NW, EPW0), pad_blk], axis=1).reshape(-1)
    norm_p = jnp.concatenate(
        [norm.reshape(NW, EPW0),
         jnp.zeros((NW, PAD), jnp.float32)], axis=1).reshape(-1)
    h = _tc_in(x, W0, b0)
    x0 = h
    for i in range(Wc.shape[0]):
        parts = _sc_aggregate(h, src, dst, norm_p)
        h = _tc_layer(parts, Wc[i], bc[i], x0)
    return _tc_out(h, Wout, bout)


# speedup vs baseline: 11.3566x; 1.0046x over previous
import functools

import jax
import jax.numpy as jnp
from jax import lax
from jax.experimental import pallas as pl
from jax.experimental.pallas import tpu as pltpu
from jax.experimental.pallas import tpu_sc as plsc

N = 10000
E = 320000
D = 128

NC = 2
NS = 16
NW = NC * NS

EPW0 = E // NW
K = 112
PAD = 80
EPW = EPW0 + PAD
CH = EPW // K
NP = 10240
RPT = NP // NS


def _sc_aggregate(h, src, dst, norm):
    mesh = plsc.VectorSubcoreMesh(core_axis_name="c", subcore_axis_name="s",
                                  num_cores=NC)

    @functools.partial(
        pl.kernel, mesh=mesh,
        out_type=jax.ShapeDtypeStruct((NC, NP, D), jnp.float32),
        scratch_types=[
            pltpu.VMEM((K,), jnp.int32),
            pltpu.VMEM((K,), jnp.int32),
            pltpu.VMEM((K,), jnp.int32),
            pltpu.VMEM((K,), jnp.int32),
            pltpu.VMEM((K,), jnp.int32),
            pltpu.VMEM((K,), jnp.int32),
            pltpu.VMEM((K,), jnp.float32),
            pltpu.VMEM((K,), jnp.float32),
            pltpu.VMEM((K,), jnp.float32),
            pltpu.VMEM((K, D), jnp.float32),
            pltpu.VMEM((K, D), jnp.float32),
            pltpu.VMEM((K, D), jnp.float32),
            pltpu.VMEM_SHARED((NP, D), jnp.float32),
            pltpu.SemaphoreType.DMA,
            pltpu.SemaphoreType.DMA,
            pltpu.SemaphoreType.DMA,
            pltpu.SemaphoreType.DMA,
            pltpu.SemaphoreType.DMA,
            pltpu.SemaphoreType.DMA,
            pltpu.SemaphoreType.DMA,
            pltpu.SemaphoreType.DMA,
            pltpu.SemaphoreType.DMA,
            pltpu.SemaphoreType.DMA,
            pltpu.SemaphoreType.DMA,
            pltpu.SemaphoreType.DMA,
            pltpu.SemaphoreType.DMA,
            pltpu.SemaphoreType.DMA,
            pltpu.SemaphoreType.DMA,
        ],
    )
    def agg_kernel(h_hbm, src_hbm, dst_hbm, norm_hbm, out_hbm,
                   src0, src1, src2, dst0, dst1, dst2,
                   norm0, norm1, norm2, rows0, rows1, rows2, agg_sh,
                   ssem0, ssem1, ssem2, dsem0, dsem1, dsem2,
                   nsem0, nsem1, nsem2,
                   gsem0, gsem1, gsem2, csem0, csem1, csem2):
        c = lax.axis_index("c")
        s = lax.axis_index("s")
        wid = s * NC + c
        e0 = wid * EPW

        pltpu.async_copy(src_hbm.at[pl.ds(e0, K)], src0, ssem0)
        pltpu.async_copy(src_hbm.at[pl.ds(e0 + K, K)], src1, ssem1)
        pltpu.async_copy(src_hbm.at[pl.ds(e0 + 2 * K, K)], src2, ssem2)
        pltpu.async_copy(dst_hbm.at[pl.ds(e0, K)], dst0, dsem0)
        pltpu.async_copy(dst_hbm.at[pl.ds(e0 + K, K)], dst1, dsem1)
        pltpu.async_copy(dst_hbm.at[pl.ds(e0 + 2 * K, K)], dst2, dsem2)
        pltpu.async_copy(norm_hbm.at[pl.ds(e0, K)], norm0, nsem0)
        pltpu.async_copy(norm_hbm.at[pl.ds(e0 + K, K)], norm1, nsem1)
        pltpu.async_copy(norm_hbm.at[pl.ds(e0 + 2 * K, K)], norm2, nsem2)

        zero16 = jnp.zeros((16,), jnp.float32)

        def zrow(r, _):
            for j in range(D // 16):
                rows0[r, pl.ds(j * 16, 16)] = zero16
            return 0

        lax.fori_loop(0, 80, zrow, 0)

        def zcopy(t, _):
            pltpu.sync_copy(rows0.at[pl.ds(0, 80)],
                            agg_sh.at[pl.ds(s * RPT + t * 80, 80)])
            return 0

        lax.fori_loop(0, RPT // 80, zcopy, 0)
        plsc.subcore_barrier()

        pltpu.make_async_copy(src_hbm.at[pl.ds(e0, K)], src0, ssem0).wait()
        pltpu.async_copy(h_hbm.at[src0], rows0, gsem0)

        bufs = ((src0, ssem0, dst0, dsem0, norm0, nsem0, rows0, gsem0, csem0),
                (src1, ssem1, dst1, dsem1, norm1, nsem1, rows1, gsem1, csem1),
                (src2, ssem2, dst2, dsem2, norm2, nsem2, rows2, gsem2, csem2))

        def triple(t, _):
            for r in range(3):
                i = 3 * t + r
                (src_c, ssem_c, dst_c, dsem_c, norm_c, nsem_c,
                 rows_c, gsem_c, csem_c) = bufs[r]
                (src_n, ssem_n, dst_n, dsem_n, norm_n, nsem_n,
                 rows_n, gsem_n, csem_n) = bufs[(r + 1) % 3]
                (src_n2, ssem_n2, dst_n2, dsem_n2, norm_n2, nsem_n2,
                 rows_n2, gsem_n2, csem_n2) = bufs[(r + 2) % 3]

                @pl.when(i + 1 < CH)
                def _():
                    pltpu.make_async_copy(
                        src_hbm.at[pl.ds(e0 + (i + 1) * K, K)],
                        src_n, ssem_n).wait()
                    pltpu.async_copy(h_hbm.at[src_n], rows_n, gsem_n)

                @pl.when(i < CH)
                def _():
                    pltpu.make_async_copy(
                        norm_hbm.at[pl.ds(e0 + i * K, K)],
                        norm_c, nsem_c).wait()
                    pltpu.make_async_copy(
                        h_hbm.at[src_c], rows_c, gsem_c).wait()

                    def edge16(g, _):
                        nv = norm_c[pl.ds(g * 16, 16)]
                        for l in range(16):
                            k = g * 16 + l
                            nk = nv[l]
                            for j in range(D // 16):
                                sl = pl.ds(j * 16, 16)
                                rows_c[k, sl] = rows_c[k, sl] * nk
                        return 0

                    lax.fori_loop(0, K // 16, edge16, 0, unroll=2)
                    pltpu.make_async_copy(
                        dst_hbm.at[pl.ds(e0 + i * K, K)],
                        dst_c, dsem_c).wait()
                    pltpu.async_copy(rows_c, agg_sh.at[dst_c], csem_c,
                                     add=True)

                @pl.when(jnp.logical_and(i >= 1, i + 2 < CH))
                def _():
                    pltpu.make_async_copy(
                        rows_n2, agg_sh.at[dst_n2], csem_n2).wait()
                    pltpu.async_copy(
                        src_hbm.at[pl.ds(e0 + (i + 2) * K, K)],
                        src_n2, ssem_n2)
                    pltpu.async_copy(
                        dst_hbm.at[pl.ds(e0 + (i + 2) * K, K)],
                        dst_n2, dsem_n2)
                    pltpu.async_copy(
                        norm_hbm.at[pl.ds(e0 + (i + 2) * K, K)],
                        norm_n2, nsem_n2)

            return 0

        lax.fori_loop(0, (CH + 2) // 3, triple, 0)

        pltpu.make_async_copy(rows0, agg_sh.at[dst0], csem0).wait()
        pltpu.make_async_copy(rows1, agg_sh.at[dst1], csem1).wait()
        pltpu.make_async_copy(rows2, agg_sh.at[dst2], csem2).wait()
        plsc.subcore_barrier()

        pltpu.sync_copy(agg_sh.at[pl.ds(s * RPT, RPT)],
                        out_hbm.at[c, pl.ds(s * RPT, RPT)])

    return agg_kernel(h, src, dst, norm)


_R = 1000


def _tc_in_kernel(x_ref, w_ref, b_ref, o_ref):
    z = jnp.dot(x_ref[...], w_ref[...], preferred_element_type=jnp.float32)
    o_ref[...] = jnp.maximum(z + b_ref[...], 0.0)


def _tc_in(x, W0, b0):
    return pl.pallas_call(
        _tc_in_kernel,
        grid=(N // _R,),
        in_specs=[
            pl.BlockSpec((_R, D), lambda i: (i, 0)),
            pl.BlockSpec((D, D), lambda i: (0, 0)),
            pl.BlockSpec((1, D), lambda i: (0, 0)),
        ],
        out_specs=pl.BlockSpec((_R, D), lambda i: (i, 0)),
        out_shape=jax.ShapeDtypeStruct((N, D), jnp.float32),
    )(x, W0, b0.reshape(1, D))


def _tc_layer_kernel(p_ref, w_ref, b_ref, x0_ref, o_ref):
    a = p_ref[0] + p_ref[1]
    z = jnp.dot(a, w_ref[...], preferred_element_type=jnp.float32)
    o_ref[...] = jnp.maximum(z + b_ref[...] + x0_ref[...], 0.0)


def _tc_layer(parts, W, b, x0):
    return pl.pallas_call(
        _tc_layer_kernel,
        grid=(N // _R,),
        in_specs=[
            pl.BlockSpec((NC, _R, D), lambda i: (0, i, 0)),
            pl.BlockSpec((D, D), lambda i: (0, 0)),
            pl.BlockSpec((1, D), lambda i: (0, 0)),
            pl.BlockSpec((_R, D), lambda i: (i, 0)),
        ],
        out_specs=pl.BlockSpec((_R, D), lambda i: (i, 0)),
        out_shape=jax.ShapeDtypeStruct((N, D), jnp.float32),
    )(parts, W, b.reshape(1, D), x0)


def _tc_last_kernel(p_ref, w_ref, b_ref, x0_ref, wo_ref, bo_ref, o_ref):
    a = p_ref[0] + p_ref[1]
    hh = jnp.dot(a, w_ref[...], preferred_element_type=jnp.float32)
    hh = jnp.maximum(hh + b_ref[...] + x0_ref[...], 0.0)
    z = jnp.dot(hh, wo_ref[...], preferred_element_type=jnp.float32)
    z = z + bo_ref[...]
    m = jnp.max(z, axis=1, keepdims=True)
    lse = jnp.log(jnp.sum(jnp.exp(z - m), axis=1, keepdims=True))
    o_ref[...] = z - m - lse


def _tc_last(parts, W, b, x0, Wout, bout):
    dout = Wout.shape[1]
    return pl.pallas_call(
        _tc_last_kernel,
        grid=(N // _R,),
        in_specs=[
            pl.BlockSpec((NC, _R, D), lambda i: (0, i, 0)),
            pl.BlockSpec((D, D), lambda i: (0, 0)),
            pl.BlockSpec((1, D), lambda i: (0, 0)),
            pl.BlockSpec((_R, D), lambda i: (i, 0)),
            pl.BlockSpec((D, dout), lambda i: (0, 0)),
            pl.BlockSpec((1, dout), lambda i: (0, 0)),
        ],
        out_specs=pl.BlockSpec((_R, dout), lambda i: (i, 0)),
        out_shape=jax.ShapeDtypeStruct((N, dout), jnp.float32),
    )(parts, W, b.reshape(1, D), x0, Wout, bout.reshape(1, dout))


def kernel(x, edge_index, norm, W0, b0, Wc, bc, Wout, bout):
    spread = (jnp.arange(PAD, dtype=jnp.int32) * 911) % N
    pad_blk = jnp.broadcast_to(spread, (NW, PAD))
    src = jnp.concatenate(
        [edge_index[0].reshape(NW, EPW0), pad_blk], axis=1).reshape(-1)
    dst = jnp.concatenate(
        [edge_index[1].reshape(NW, EPW0), pad_blk], axis=1).reshape(-1)
    norm_p = jnp.concatenate(
        [norm.reshape(NW, EPW0),
         jnp.zeros((NW, PAD), jnp.float32)], axis=1).reshape(-1)
    h = _tc_in(x, W0, b0)
    x0 = h
    L = Wc.shape[0]
    for i in range(L - 1):
        parts = _sc_aggregate(h, src, dst, norm_p)
        h = _tc_layer(parts, Wc[i], bc[i], x0)
    parts = _sc_aggregate(h, src, dst, norm_p)
    return _tc_last(parts, Wc[L - 1], bc[L - 1], x0, Wout, bout)
